# baseline (speedup 1.0000x reference)
import functools

import jax
import jax.numpy as jnp
from jax.experimental import pallas as pl

_N = 10000
_D = 256
_K = 3
_BN = 400


def _mlp_body(h_ref, w_ref, b_ref, o_ref, *, relu):
    acc = jnp.dot(h_ref[...], w_ref[...], preferred_element_type=jnp.float32)
    acc = acc + b_ref[...]
    if relu:
        acc = jnp.maximum(acc, 0.0)
    o_ref[...] = acc


def _fused_matmul(H, Wc, b, relu):
    n, kdim = H.shape
    dout = Wc.shape[1]
    grid = (n // _BN,)
    return pl.pallas_call(
        functools.partial(_mlp_body, relu=relu),
        grid=grid,
        in_specs=[
            pl.BlockSpec((_BN, kdim), lambda i: (i, 0)),
            pl.BlockSpec((kdim, dout), lambda i: (0, 0)),
            pl.BlockSpec((1, dout), lambda i: (0, 0)),
        ],
        out_specs=pl.BlockSpec((_BN, dout), lambda i: (i, 0)),
        out_shape=jax.ShapeDtypeStruct((n, dout), jnp.float32),
    )(H, Wc, b.reshape(1, dout))


def kernel(x, edge_index, edge_type, W1, b1, W2, b2, W3, b3):
    row, col = edge_index[0], edge_index[1]
    n = x.shape[0]
    deg = jnp.zeros((n,), jnp.float32).at[col].add(jnp.ones_like(col, jnp.float32))
    dis = jnp.where(deg > 0, jax.lax.rsqrt(jnp.where(deg > 0, deg, 1.0)), 0.0)
    norm = dis[row] * dis[col]

    def prop(h):
        msgs = h[row] * norm[:, None]
        return jnp.zeros_like(h).at[col].add(msgs)

    def layer(h, W, b, relu):
        hops = [h]
        for _ in range(_K):
            hops.append(prop(hops[-1]))
        H = jnp.concatenate(hops, axis=1)
        Wc = W.reshape((_K + 1) * W.shape[1], W.shape[2])
        return _fused_matmul(H, Wc, b, relu)

    h = layer(x, W1, b1, True)
    h = layer(h, W2, b2, True)
    h = layer(h, W3, b3, False)
    return h


# speedup vs baseline: 6.0150x; 6.0150x over previous
import functools

import jax
import jax.numpy as jnp
from jax import lax
from jax.experimental import pallas as pl
from jax.experimental.pallas import tpu as pltpu
from jax.experimental.pallas import tpu_sc as plsc

_N = 10000
_E = 160000
_D = 256
_DH = 128
_K = 3

_NP = 10240
_NT = 16
_EPT = _E // _NT
_CB = 100
_CHUNKS = _EPT // _CB
_RPT = _NP // _NT

_BN = 400


def _sc_pass_body(g_ref, rows_ref, cols_ref, z_ref, out_ref,
                  rows_v, cols_v, buf0, buf1, acc, sem0, sem1):
    c = lax.axis_index("c")
    s = lax.axis_index("s")
    pltpu.sync_copy(z_ref, acc.at[pl.ds(s * _RPT, _RPT)])
    pltpu.sync_copy(rows_ref.at[c, s], rows_v)
    pltpu.sync_copy(cols_ref.at[s], cols_v)
    plsc.subcore_barrier()

    def step(g, carry):
        j0 = 2 * g
        j1 = j0 + 1
        h0 = pltpu.async_copy(g_ref.at[rows_v.at[j0]], buf0, sem0)
        h1 = pltpu.async_copy(g_ref.at[rows_v.at[j1]], buf1, sem1)
        h0.wait()
        pltpu.sync_copy(buf0, acc.at[cols_v.at[j0]], add=True)
        h1.wait()
        pltpu.sync_copy(buf1, acc.at[cols_v.at[j1]], add=True)
        return carry

    lax.fori_loop(0, _CHUNKS // 2, step, 0)
    plsc.subcore_barrier()
    pltpu.sync_copy(acc.at[pl.ds(s * _RPT, _RPT)],
                    out_ref.at[c, pl.ds(s * _RPT, _RPT)])


_sc_pass = functools.partial(
    pl.kernel,
    _sc_pass_body,
    out_type=jax.ShapeDtypeStruct((2, _NP, _DH), jnp.float32),
    mesh=plsc.VectorSubcoreMesh(core_axis_name="c", subcore_axis_name="s"),
    scratch_types=[
        pltpu.VMEM((_CHUNKS, _CB), jnp.int32),
        pltpu.VMEM((_CHUNKS, _CB), jnp.int32),
        pltpu.VMEM((_CB, _DH), jnp.float32),
        pltpu.VMEM((_CB, _DH), jnp.float32),
        pltpu.VMEM_SHARED((_NP, _DH), jnp.float32),
        pltpu.SemaphoreType.DMA,
        pltpu.SemaphoreType.DMA,
    ],
    compiler_params=pltpu.CompilerParams(use_tc_tiling_on_sc=False),
)()


def _scale_split_body(h_ref, d_ref, o_ref):
    d = d_ref[...]
    o_ref[0] = h_ref[:, :_DH] * d
    o_ref[1] = h_ref[:, _DH:] * d


def _scale_split(h, d):
    return pl.pallas_call(
        _scale_split_body,
        grid=(_N // _BN,),
        in_specs=[
            pl.BlockSpec((_BN, _D), lambda i: (i, 0)),
            pl.BlockSpec((_BN, 1), lambda i: (i, 0)),
        ],
        out_specs=pl.BlockSpec((2, _BN, _DH), lambda i: (0, i, 0)),
        out_shape=jax.ShapeDtypeStruct((2, _NP, _DH), jnp.float32),
    )(h, d)


def _scale2_body(p_ref, d_ref, o_ref):
    o_ref[...] = p_ref[...] * d_ref[...]


def _scale2(p, d2):
    return pl.pallas_call(
        _scale2_body,
        grid=(2, _N // _BN),
        in_specs=[
            pl.BlockSpec((1, _BN, _DH), lambda c, i: (c, i, 0)),
            pl.BlockSpec((1, _BN, 1), lambda c, i: (0, i, 0)),
        ],
        out_specs=pl.BlockSpec((1, _BN, _DH), lambda c, i: (c, i, 0)),
        out_shape=jax.ShapeDtypeStruct((2, _NP, _DH), jnp.float32),
    )(p, d2[None])


def _tag_mm_body(h_ref, p1a, p1b, p2a, p2b, p3a, p3b, d_ref,
                 w_ref, b_ref, o_ref, *, relu):
    d = d_ref[...]
    acc = jnp.dot(h_ref[...], w_ref[0], preferred_element_type=jnp.float32)
    for k, (pa, pb) in enumerate(((p1a, p1b), (p2a, p2b), (p3a, p3b))):
        w = w_ref[k + 1]
        acc += jnp.dot(pa[0] * d, w[:_DH, :], preferred_element_type=jnp.float32)
        acc += jnp.dot(pb[0] * d, w[_DH:, :], preferred_element_type=jnp.float32)
    acc += b_ref[...]
    if relu:
        acc = jnp.maximum(acc, 0.0)
    o_ref[...] = acc


def _tag_mm(h, p1, p2, p3, d, W, b, relu):
    pspec_a = pl.BlockSpec((1, _BN, _DH), lambda i: (0, i, 0))
    pspec_b = pl.BlockSpec((1, _BN, _DH), lambda i: (1, i, 0))
    return pl.pallas_call(
        functools.partial(_tag_mm_body, relu=relu),
        grid=(_N // _BN,),
        in_specs=[
            pl.BlockSpec((_BN, _D), lambda i: (i, 0)),
            pspec_a, pspec_b, pspec_a, pspec_b, pspec_a, pspec_b,
            pl.BlockSpec((_BN, 1), lambda i: (i, 0)),
            pl.BlockSpec((_K + 1, _D, _D), lambda i: (0, 0, 0)),
            pl.BlockSpec((1, _D), lambda i: (0, 0)),
        ],
        out_specs=pl.BlockSpec((_BN, _D), lambda i: (i, 0)),
        out_shape=jax.ShapeDtypeStruct((_N, _D), jnp.float32),
    )(h, p1, p1, p2, p2, p3, p3, d, W, b.reshape(1, _D))


def kernel(x, edge_index, edge_type, W1, b1, W2, b2, W3, b3):
    row, col = edge_index[0], edge_index[1]
    deg = jnp.zeros((_N,), jnp.float32).at[col].add(
        jnp.ones_like(col, jnp.float32))
    dis = jnp.where(deg > 0, lax.rsqrt(jnp.where(deg > 0, deg, 1.0)), 0.0)
    d = dis[:, None]
    d2 = (dis * dis)[:, None]

    rows4 = jnp.stack([row, row + _NP]).reshape(2, _NT, _CHUNKS, _CB)
    cols3 = col.reshape(_NT, _CHUNKS, _CB)
    zeros = jnp.zeros((_RPT, _DH), jnp.float32)

    def prop(g):
        return _sc_pass(g.reshape(2 * _NP, _DH), rows4, cols3, zeros)

    def layer(h, W, b, relu):
        g = _scale_split(h, d)
        p1 = prop(g)
        p2 = prop(_scale2(p1, d2))
        p3 = prop(_scale2(p2, d2))
        return _tag_mm(h, p1, p2, p3, d, W, b, relu)

    h = layer(x, W1, b1, True)
    h = layer(h, W2, b2, True)
    h = layer(h, W3, b3, False)
    return h


# speedup vs baseline: 6.4163x; 1.0667x over previous
import functools

import jax
import jax.numpy as jnp
from jax import lax
from jax.experimental import pallas as pl
from jax.experimental.pallas import tpu as pltpu
from jax.experimental.pallas import tpu_sc as plsc

_N = 10000
_E = 160000
_D = 256
_DH = 128
_K = 3

_NP = 10240
_NT = 16
_EPT = _E // _NT
_CB = 100
_CHUNKS = _EPT // _CB
_RPT = _NP // _NT

_BN = 400


def _sc_pass_body(g_ref, rows_ref, cols_ref, z_ref, out_ref,
                  rows_v, cols_v, buf0, buf1, acc,
                  gsem0, gsem1, ssem0, ssem1):
    c = lax.axis_index("c")
    s = lax.axis_index("s")
    pltpu.sync_copy(z_ref, acc.at[pl.ds(s * _RPT, _RPT)])
    pltpu.sync_copy(rows_ref.at[c, s], rows_v)
    pltpu.sync_copy(cols_ref.at[s], cols_v)
    pltpu.async_copy(g_ref.at[rows_v.at[0]], buf0, gsem0)
    plsc.subcore_barrier()

    def wait_g(buf, sem, j):
        pltpu.make_async_copy(g_ref.at[rows_v.at[j]], buf, sem).wait()

    def wait_s(buf, sem, j):
        pltpu.make_async_copy(buf, acc.at[cols_v.at[j]], sem).wait()

    def step(g, carry):
        j0 = 2 * g
        j1 = j0 + 1
        wait_g(buf0, gsem0, j0)
        pltpu.async_copy(buf0, acc.at[cols_v.at[j0]], ssem0, add=True)

        @pl.when(g > 0)
        def _():
            wait_s(buf1, ssem1, j0)
        pltpu.async_copy(g_ref.at[rows_v.at[j1]], buf1, gsem1)
        wait_g(buf1, gsem1, j1)
        pltpu.async_copy(buf1, acc.at[cols_v.at[j1]], ssem1, add=True)
        wait_s(buf0, ssem0, j0)

        @pl.when(g < _CHUNKS // 2 - 1)
        def _():
            pltpu.async_copy(g_ref.at[rows_v.at[j0 + 2]], buf0, gsem0)
        return carry

    lax.fori_loop(0, _CHUNKS // 2, step, 0)
    wait_s(buf1, ssem1, 0)
    plsc.subcore_barrier()
    pltpu.sync_copy(acc.at[pl.ds(s * _RPT, _RPT)],
                    out_ref.at[c, pl.ds(s * _RPT, _RPT)])


_sc_pass = functools.partial(
    pl.kernel,
    _sc_pass_body,
    out_type=jax.ShapeDtypeStruct((2, _NP, _DH), jnp.float32),
    mesh=plsc.VectorSubcoreMesh(core_axis_name="c", subcore_axis_name="s"),
    scratch_types=[
        pltpu.VMEM((_CHUNKS, _CB), jnp.int32),
        pltpu.VMEM((_CHUNKS, _CB), jnp.int32),
        pltpu.VMEM((_CB, _DH), jnp.float32),
        pltpu.VMEM((_CB, _DH), jnp.float32),
        pltpu.VMEM_SHARED((_NP, _DH), jnp.float32),
        pltpu.SemaphoreType.DMA,
        pltpu.SemaphoreType.DMA,
        pltpu.SemaphoreType.DMA,
        pltpu.SemaphoreType.DMA,
    ],
    compiler_params=pltpu.CompilerParams(use_tc_tiling_on_sc=False),
)()


def _scale_split_body(h_ref, d_ref, o_ref):
    d = d_ref[...]
    o_ref[0] = h_ref[:, :_DH] * d
    o_ref[1] = h_ref[:, _DH:] * d


def _scale_split(h, d):
    return pl.pallas_call(
        _scale_split_body,
        grid=(_N // _BN,),
        in_specs=[
            pl.BlockSpec((_BN, _D), lambda i: (i, 0)),
            pl.BlockSpec((_BN, 1), lambda i: (i, 0)),
        ],
        out_specs=pl.BlockSpec((2, _BN, _DH), lambda i: (0, i, 0)),
        out_shape=jax.ShapeDtypeStruct((2, _NP, _DH), jnp.float32),
    )(h, d)


def _scale2_body(p_ref, d_ref, o_ref):
    o_ref[...] = p_ref[...] * d_ref[...]


def _scale2(p, d2):
    return pl.pallas_call(
        _scale2_body,
        grid=(2, _N // _BN),
        in_specs=[
            pl.BlockSpec((1, _BN, _DH), lambda c, i: (c, i, 0)),
            pl.BlockSpec((1, _BN, 1), lambda c, i: (0, i, 0)),
        ],
        out_specs=pl.BlockSpec((1, _BN, _DH), lambda c, i: (c, i, 0)),
        out_shape=jax.ShapeDtypeStruct((2, _NP, _DH), jnp.float32),
    )(p, d2[None])


def _tag_mm_body(h_ref, p1a, p1b, p2a, p2b, p3a, p3b, d_ref,
                 w_ref, b_ref, o_ref, *, relu):
    d = d_ref[...]
    acc = jnp.dot(h_ref[...], w_ref[0], preferred_element_type=jnp.float32)
    for k, (pa, pb) in enumerate(((p1a, p1b), (p2a, p2b), (p3a, p3b))):
        w = w_ref[k + 1]
        acc += jnp.dot(pa[0] * d, w[:_DH, :], preferred_element_type=jnp.float32)
        acc += jnp.dot(pb[0] * d, w[_DH:, :], preferred_element_type=jnp.float32)
    acc += b_ref[...]
    if relu:
        acc = jnp.maximum(acc, 0.0)
    o_ref[...] = acc


def _tag_mm(h, p1, p2, p3, d, W, b, relu):
    pspec_a = pl.BlockSpec((1, _BN, _DH), lambda i: (0, i, 0))
    pspec_b = pl.BlockSpec((1, _BN, _DH), lambda i: (1, i, 0))
    return pl.pallas_call(
        functools.partial(_tag_mm_body, relu=relu),
        grid=(_N // _BN,),
        in_specs=[
            pl.BlockSpec((_BN, _D), lambda i: (i, 0)),
            pspec_a, pspec_b, pspec_a, pspec_b, pspec_a, pspec_b,
            pl.BlockSpec((_BN, 1), lambda i: (i, 0)),
            pl.BlockSpec((_K + 1, _D, _D), lambda i: (0, 0, 0)),
            pl.BlockSpec((1, _D), lambda i: (0, 0)),
        ],
        out_specs=pl.BlockSpec((_BN, _D), lambda i: (i, 0)),
        out_shape=jax.ShapeDtypeStruct((_N, _D), jnp.float32),
    )(h, p1, p1, p2, p2, p3, p3, d, W, b.reshape(1, _D))


def kernel(x, edge_index, edge_type, W1, b1, W2, b2, W3, b3):
    row, col = edge_index[0], edge_index[1]
    deg = jnp.zeros((_N,), jnp.float32).at[col].add(
        jnp.ones_like(col, jnp.float32))
    dis = jnp.where(deg > 0, lax.rsqrt(jnp.where(deg > 0, deg, 1.0)), 0.0)
    d = dis[:, None]
    d2 = (dis * dis)[:, None]

    rows4 = jnp.stack([row, row + _NP]).reshape(2, _NT, _CHUNKS, _CB)
    cols3 = col.reshape(_NT, _CHUNKS, _CB)
    zeros = jnp.zeros((_RPT, _DH), jnp.float32)

    def prop(g):
        return _sc_pass(g.reshape(2 * _NP, _DH), rows4, cols3, zeros)

    def layer(h, W, b, relu):
        g = _scale_split(h, d)
        p1 = prop(g)
        p2 = prop(_scale2(p1, d2))
        p3 = prop(_scale2(p2, d2))
        return _tag_mm(h, p1, p2, p3, d, W, b, relu)

    h = layer(x, W1, b1, True)
    h = layer(h, W2, b2, True)
    h = layer(h, W3, b3, False)
    return h
